# baseline (speedup 1.0000x reference)
import functools

import jax
import jax.numpy as jnp
from jax import lax
from jax.experimental import pallas as pl
from jax.experimental.pallas import tpu as pltpu
from jax.experimental.pallas import tpu_sc as plsc


_IMGS = 384
_NC = 2
_NS = 16
_NW = _NC * _NS
_PER_W = _IMGS // _NW
_STEP = 2


def _sc_copy(x_hbm, o_hbm, buf):
    wid = lax.axis_index("s") * _NC + lax.axis_index("c")
    base = wid * _PER_W
    for j in range(_PER_W // _STEP):
        pltpu.sync_copy(x_hbm.at[pl.ds(base + j * _STEP, _STEP)], buf)
        pltpu.sync_copy(buf, o_hbm.at[pl.ds(base + j * _STEP, _STEP)])


def kernel(feature_batch, box_batch):
    x = feature_batch.reshape(_IMGS, 224, 224)
    mesh = plsc.VectorSubcoreMesh(core_axis_name="c", subcore_axis_name="s")
    fn = functools.partial(
        pl.kernel,
        out_type=jax.ShapeDtypeStruct((_IMGS, 224, 224), jnp.float32),
        mesh=mesh,
        scratch_types=[pltpu.VMEM((_STEP, 224, 224), jnp.float32)],
    )(_sc_copy)
    out = fn(x)
    return out.reshape(feature_batch.shape)


# speedup vs baseline: 1.4782x; 1.4782x over previous
import functools
import math

import jax
import jax.numpy as jnp
from jax import lax
from jax.experimental import pallas as pl
from jax.experimental.pallas import tpu as pltpu


_IMGS = 384
_NCH = 6
_SLOTS = 3


def _make_relay(ncores):
    share = _IMGS // ncores
    ch = share // _NCH

    def relay(x_hbm, o_hbm, buf, in_sems, out_sems):
        base = lax.axis_index("core") * share

        def cin(i, slot):
            return pltpu.make_async_copy(
                x_hbm.at[pl.ds(base + i * ch, ch)], buf.at[slot],
                in_sems.at[slot])

        def cout(i, slot):
            return pltpu.make_async_copy(
                buf.at[slot], o_hbm.at[pl.ds(base + i * ch, ch)],
                out_sems.at[slot])

        for k in range(_SLOTS):
            cin(k, k).start()
        pending = None
        for i in range(_NCH):
            s = i % _SLOTS
            if pending is not None:
                pj, ps = pending
                cout(pj, ps).wait()
                cin(pj + _SLOTS, ps).start()
                pending = None
            cin(i, s).wait()
            cout(i, s).start()
            if i + _SLOTS < _NCH:
                pending = (i, s)
        for i in range(_NCH - _SLOTS, _NCH):
            cout(i, i % _SLOTS).wait()

    return relay, ch


def kernel(feature_batch, box_batch):
    x = feature_batch.reshape(_IMGS, 224, 224)
    mesh = pltpu.create_tensorcore_mesh("core")
    ncores = math.prod(mesh.devices.shape)
    relay, ch = _make_relay(ncores)
    fn = functools.partial(
        pl.kernel,
        out_type=jax.ShapeDtypeStruct((_IMGS, 224, 224), jnp.float32),
        mesh=mesh,
        scratch_types=[
            pltpu.VMEM((_SLOTS, ch, 224, 224), jnp.float32),
            pltpu.SemaphoreType.DMA((_SLOTS,)),
            pltpu.SemaphoreType.DMA((_SLOTS,)),
        ],
    )(relay)
    out = fn(x)
    return out.reshape(feature_batch.shape)
